# Initial kernel scaffold; baseline (speedup 1.0000x reference)
import jax
import jax.numpy as jnp
from jax.experimental import pallas as pl


def kernel(x, edge_index, W_s2d, b_s2d, W_d2s, b_d2s, W1_ii, b1_ii, W2_ii, b2_ii, W1_oo, b1_oo, W2_oo, b2_oo, W1_io, b1_io, W2_io, b2_io, W1_oi, b1_oi, W2_oi, b2_oi):
    raise NotImplementedError("write your pallas kernel here")


# speedup vs baseline: 10.0094x; 10.0094x over previous
import functools

import jax
import jax.numpy as jnp
from jax import lax
from jax.experimental import pallas as pl
from jax.experimental.pallas import tpu as pltpu
from jax.experimental.pallas import tpu_sc as plsc

N = 10000
E = 160000
D = 256
H = 128
NC = 2
NT = 16
NP = 10240
RPT = NP // NT
EPT = 10240
CH = EPT // 128
F32 = jnp.float32

_mesh = plsc.VectorSubcoreMesh(core_axis_name="c", subcore_axis_name="s")
_sc_params = pltpu.CompilerParams(needs_layout_passes=False)


@functools.partial(
    pl.kernel,
    out_type=jax.ShapeDtypeStruct((NC, NT, NP), F32),
    mesh=_mesh,
    compiler_params=_sc_params,
    scratch_types=[
        pltpu.VMEM((CH, 128), jnp.int32),
        pltpu.VMEM((NP,), F32),
    ],
)
def _sc_degrees(eidx_hbm, out_hbm, idx_v, deg_v):
    c = lax.axis_index("c")
    t = lax.axis_index("s")
    pltpu.sync_copy(eidx_hbm.at[0, c, t], idx_v)

    zero16 = jnp.zeros((16,), F32)
    one16 = jnp.ones((16,), F32)

    def _zero(i, _):
        deg_v[pl.ds(i * 16, 16)] = zero16
        return ()

    lax.fori_loop(0, NP // 16, _zero, ())

    def _hist(j, _):
        for k in range(8):
            idx = idx_v[j, pl.ds(k * 16, 16)]
            plsc.addupdate_scatter(deg_v, [idx], one16)
        return ()

    lax.fori_loop(0, CH, _hist, ())
    pltpu.sync_copy(deg_v, out_hbm.at[c, t])


def _tc_prescale_body(dsum_ref, x_ref, W1s_ref, W2s_ref, b1s_ref, ball_ref,
                      b2s_ref, u_ref, v_ref, sdis_ref, disc_ref, disc2_ref,
                      Wc_ref, bvec_ref):
    d = jnp.sum(dsum_ref[...], axis=1)
    dis = jnp.where(d > 0, lax.rsqrt(d), 0.0)
    dis_r = dis[0][:, None]
    dis_c = dis[1][:, None]
    xb = x_ref[...]
    u = dis_c * xb
    v = dis_r * xb
    u_ref[0] = u[:, :H]
    u_ref[1] = u[:, H:]
    v_ref[0] = v[:, :H]
    v_ref[1] = v[:, H:]
    pad = jnp.zeros((xb.shape[0], 14), F32)
    sdis_ref[...] = jnp.concatenate([dis_c, dis_r, pad], axis=1)
    disc_ref[...] = dis[1]
    disc2_ref[...] = dis[1] * dis[1]

    @pl.when(pl.program_id(0) == 0)
    def _():
        wc = jnp.zeros((D, D), F32)
        bc1 = jnp.zeros((1, D), F32)
        for m in range(4):
            w2 = W2s_ref[m]
            wc = wc + jnp.dot(W1s_ref[m], w2,
                              preferred_element_type=F32,
                              precision=lax.Precision.HIGHEST)
            bc1 = bc1 + jnp.dot(b1s_ref[m][None, :], w2,
                                preferred_element_type=F32,
                                precision=lax.Precision.HIGHEST)
        Wc_ref[...] = wc
        btot = jnp.sum(ball_ref[...], axis=0) + jnp.sum(b2s_ref[...], axis=0)
        bvec_ref[0:1, :] = bc1
        bvec_ref[1:2, :] = btot[None, :]


_BLK_B = 256


def _tc_prescale(dsum, xp, W1s, W2s, b1s, ball, b2s):
    nb = NP // _BLK_B
    full = lambda *shape: pl.BlockSpec(shape, lambda i: tuple(0 for _ in shape))
    return pl.pallas_call(
        _tc_prescale_body,
        grid=(nb,),
        in_specs=[
            pl.BlockSpec((NC, NT, _BLK_B), lambda i: (0, 0, i)),
            pl.BlockSpec((_BLK_B, D), lambda i: (i, 0)),
            full(4, D, D),
            full(4, D, D),
            full(4, D),
            full(2, D),
            full(4, D),
        ],
        out_specs=[
            pl.BlockSpec((NC, _BLK_B, H), lambda i: (0, i, 0)),
            pl.BlockSpec((NC, _BLK_B, H), lambda i: (0, i, 0)),
            pl.BlockSpec((_BLK_B, 16), lambda i: (i, 0)),
            pl.BlockSpec((_BLK_B,), lambda i: (i,)),
            pl.BlockSpec((_BLK_B,), lambda i: (i,)),
            pl.BlockSpec((D, D), lambda i: (0, 0)),
            pl.BlockSpec((2, D), lambda i: (0, 0)),
        ],
        out_shape=[
            jax.ShapeDtypeStruct((NC, NP, H), F32),
            jax.ShapeDtypeStruct((NC, NP, H), F32),
            jax.ShapeDtypeStruct((NP, 16), F32),
            jax.ShapeDtypeStruct((NP,), F32),
            jax.ShapeDtypeStruct((NP,), F32),
            jax.ShapeDtypeStruct((D, D), F32),
            jax.ShapeDtypeStruct((2, D), F32),
        ],
    )(dsum, xp, W1s, W2s, b1s, ball, b2s)


@functools.partial(
    pl.kernel,
    out_type=jax.ShapeDtypeStruct((NC, NT, NP), F32),
    mesh=_mesh,
    compiler_params=_sc_params,
    scratch_types=[
        pltpu.VMEM((CH // 2, 128), jnp.int32),
        pltpu.VMEM((CH // 2, 128), jnp.int32),
        pltpu.VMEM((NP,), F32),
        pltpu.VMEM((NP,), F32),
    ],
)
def _sc_spass(eidx_hbm, disc_hbm, out_hbm, ridx_v, cidx_v, disc_v, sacc_v):
    c = lax.axis_index("c")
    t = lax.axis_index("s")
    j0 = c * (CH // 2)
    pltpu.sync_copy(eidx_hbm.at[0, 0, t, pl.ds(j0, CH // 2)], ridx_v)
    pltpu.sync_copy(eidx_hbm.at[0, 1, t, pl.ds(j0, CH // 2)], cidx_v)
    pltpu.sync_copy(disc_hbm, disc_v)

    zero16 = jnp.zeros((16,), F32)

    def _zero(i, _):
        sacc_v[pl.ds(i * 16, 16)] = zero16
        return ()

    lax.fori_loop(0, NP // 16, _zero, ())

    def _hist(j, _):
        for k in range(8):
            ridx = ridx_v[j, pl.ds(k * 16, 16)]
            cidx = cidx_v[j, pl.ds(k * 16, 16)]
            vals = plsc.load_gather(disc_v, [ridx])
            plsc.addupdate_scatter(sacc_v, [cidx], vals)
        return ()

    lax.fori_loop(0, CH // 2, _hist, ())
    pltpu.sync_copy(sacc_v, out_hbm.at[c, t])


@functools.partial(
    pl.kernel,
    out_type=(
        jax.ShapeDtypeStruct((NC * NP, H), F32),
        jax.ShapeDtypeStruct((NC * NP, H), F32),
        jax.ShapeDtypeStruct((NC * NP, H), F32),
        jax.ShapeDtypeStruct((NC * NP, H), F32),
    ),
    mesh=_mesh,
    compiler_params=_sc_params,
    scratch_types=[
        pltpu.VMEM((2, 128), jnp.int32),
        pltpu.VMEM((2, 128), jnp.int32),
        pltpu.VMEM((2, 128, H), F32),
        pltpu.VMEM((32, H), F32),
        pltpu.VMEM((RPT,), F32),
        pltpu.VMEM_SHARED((NP, H), F32),
        pltpu.SemaphoreType.DMA,
        pltpu.SemaphoreType.DMA,
        pltpu.SemaphoreType.DMA,
        pltpu.SemaphoreType.DMA,
    ],
)
def _sc_passes(eidx_hbm, u_hbm, v_hbm, disc2_hbm,
               y1_hbm, y2_hbm, z2_hbm, w4_hbm,
               ring_g, ring_s, stage, zer_v, d2_v,
               accum, semi0, semi1, semg0, semg1):
    c = lax.axis_index("c")
    t = lax.axis_index("s")
    r0 = t * RPT
    o0 = c * NP + r0
    semi = (semi0, semi1)
    semg = (semg0, semg1)

    zero16 = jnp.zeros((16,), F32)

    def _z1(i, _):
        zer_v[lax.div(i, 8), pl.ds(lax.rem(i, 8) * 16, 16)] = zero16
        return ()

    lax.fori_loop(0, 32 * H // 16, _z1, ())

    def _clear_accum():
        for k in range(RPT // 32):
            pltpu.sync_copy(zer_v, accum.at[pl.ds(r0 + k * 32, 32)])

    _clear_accum()
    pltpu.sync_copy(disc2_hbm.at[pl.ds(r0, RPT)], d2_v)

    def _run_pass(table, gplane, splane):
        def idx_load(j, b):
            pltpu.async_copy(eidx_hbm.at[c, gplane, t, j], ring_g.at[b],
                             semi[b])
            pltpu.async_copy(eidx_hbm.at[0, splane, t, j], ring_s.at[b],
                             semi[b])

        def idx_wait(j, b):
            pltpu.make_async_copy(eidx_hbm.at[c, gplane, t, j], ring_g.at[b],
                                  semi[b]).wait()
            pltpu.make_async_copy(eidx_hbm.at[0, splane, t, j], ring_s.at[b],
                                  semi[b]).wait()

        idx_load(0, 0)
        idx_load(1, 1)
        idx_wait(0, 0)
        pltpu.async_copy(table.at[ring_g.at[0]], stage.at[0], semg[0])

        def _outer(jj, _):
            for b in (0, 1):
                j = jj * 2 + b
                nb = 1 - b

                @pl.when(j + 1 < CH)
                def _():
                    idx_wait(j + 1, nb)
                    pltpu.async_copy(table.at[ring_g.at[nb]], stage.at[nb],
                                     semg[nb])

                pltpu.make_async_copy(table.at[ring_g.at[b]], stage.at[b],
                                      semg[b]).wait()
                pltpu.sync_copy(stage.at[b], accum.at[ring_s.at[b]], add=True)

                @pl.when(j + 2 < CH)
                def _():
                    idx_load(j + 2, b)
            return ()

        lax.fori_loop(0, CH // 2, _outer, ())

    plsc.subcore_barrier()
    _run_pass(u_hbm, 1, 0)
    plsc.subcore_barrier()
    pltpu.sync_copy(accum.at[pl.ds(r0, RPT)], y1_hbm.at[pl.ds(o0, RPT)])
    _clear_accum()

    plsc.subcore_barrier()
    _run_pass(v_hbm, 0, 1)
    plsc.subcore_barrier()
    pltpu.sync_copy(accum.at[pl.ds(r0, RPT)], y2_hbm.at[pl.ds(o0, RPT)])
    _clear_accum()

    plsc.subcore_barrier()
    _run_pass(u_hbm, 0, 1)
    plsc.subcore_barrier()

    for k in range(RPT // 128):
        pltpu.sync_copy(accum.at[pl.ds(r0 + k * 128, 128)], stage.at[0])

        def _grp(g, _):
            scvec = d2_v[pl.ds(k * 128 + g * 16, 16)]
            for lane in range(16):
                sc = scvec[lane]
                r = g * 16 + lane
                for q in range(H // 16):
                    sl = pl.ds(q * 16, 16)
                    stage[0, r, sl] = stage[0, r, sl] * sc
            return ()

        lax.fori_loop(0, 8, _grp, ())
        pltpu.sync_copy(stage.at[0], w4_hbm.at[pl.ds(o0 + k * 128, 128)])
    _clear_accum()

    plsc.subcore_barrier()
    _run_pass(w4_hbm, 0, 1)
    plsc.subcore_barrier()
    pltpu.sync_copy(accum.at[pl.ds(r0, RPT)], z2_hbm.at[pl.ds(o0, RPT)])


def _tc_final_body(sdis_ref, y1_ref, y2_ref, z2_ref, sp_ref,
                   Ws2d_ref, Wd2s_ref, Wc_ref, bvec_ref, out_ref):
    sd = sdis_ref[...]
    dis_c = sd[:, 0:1]
    dis_r = sd[:, 1:2]

    def mm(ref, scale, w_ref):
        a = jnp.concatenate([ref[0], ref[1]], axis=1) * scale
        return jnp.dot(a, w_ref[...], preferred_element_type=F32,
                       precision=lax.Precision.HIGHEST)

    acc = mm(y1_ref, dis_r, Ws2d_ref)
    acc = acc + mm(y2_ref, dis_c, Wd2s_ref)
    acc = acc + mm(z2_ref, dis_c, Wc_ref)
    s = dis_c * jnp.sum(sp_ref[...], axis=(0, 1))[:, None]
    acc = acc + s * bvec_ref[0:1, :] + bvec_ref[1:2, :]
    out_ref[...] = 0.75 * acc


_BLK_F = 1024


def _tc_final(sdis, y1, y2, z2, sp, Ws2d, Wd2s, Wc, bvec):
    nb = NP // _BLK_F
    full = lambda *shape: pl.BlockSpec(shape, lambda i: tuple(0 for _ in shape))
    half = pl.BlockSpec((NC, _BLK_F, H), lambda i: (0, i, 0))
    return pl.pallas_call(
        _tc_final_body,
        grid=(nb,),
        in_specs=[
            pl.BlockSpec((_BLK_F, 16), lambda i: (i, 0)),
            half, half, half,
            pl.BlockSpec((NC, NT, _BLK_F), lambda i: (0, 0, i)),
            full(D, D), full(D, D), full(D, D), full(2, D),
        ],
        out_specs=pl.BlockSpec((_BLK_F, D), lambda i: (i, 0)),
        out_shape=jax.ShapeDtypeStruct((NP, D), F32),
    )(sdis, y1, y2, z2, sp, Ws2d, Wd2s, Wc, bvec)


def kernel(x, edge_index, W_s2d, b_s2d, W_d2s, b_d2s,
           W1_ii, b1_ii, W2_ii, b2_ii,
           W1_oo, b1_oo, W2_oo, b2_oo,
           W1_io, b1_io, W2_io, b2_io,
           W1_oi, b1_oi, W2_oi, b2_oi):
    row = edge_index[0].astype(jnp.int32)
    col = edge_index[1].astype(jnp.int32)
    pad = jnp.full((NT * EPT - E,), NP - 1, jnp.int32)
    rowp = jnp.concatenate([row, pad]).reshape(NT, CH, 128)
    colp = jnp.concatenate([col, pad]).reshape(NT, CH, 128)
    base = jnp.stack([rowp, colp])
    eidx = jnp.stack([base, base + NP])

    xp = jnp.pad(x, ((0, NP - N), (0, 0)))
    W1s = jnp.stack([W1_ii, W1_oo, W1_io, W1_oi])
    W2s = jnp.stack([W2_ii, W2_oo, W2_io, W2_oi])
    b1s = jnp.stack([b1_ii, b1_oo, b1_io, b1_oi])
    b2s = jnp.stack([b2_ii, b2_oo, b2_io, b2_oi])
    ball = jnp.stack([b_s2d, b_d2s])

    dsum = _sc_degrees(eidx)
    u, v, sdis, disc, disc2, Wc, bvec = _tc_prescale(dsum, xp, W1s, W2s, b1s,
                                                     ball, b2s)
    sp = _sc_spass(eidx, disc)
    uf = u.reshape(NC * NP, H)
    vf = v.reshape(NC * NP, H)
    y1, y2, z2, _w4 = _sc_passes(eidx, uf, vf, disc2)
    sh = (NC, NP, H)
    outp = _tc_final(sdis, y1.reshape(sh), y2.reshape(sh), z2.reshape(sh), sp,
                     W_s2d, W_d2s, Wc, bvec)
    return outp[:N]
